# baseline (speedup 1.0000x reference)
import functools
import math

import jax
import jax.numpy as jnp
from jax import lax
from jax.experimental import pallas as pl
from jax.experimental.pallas import tpu as pltpu
from jax.experimental.pallas import tpu_sc as plsc

_D = 64
_LANES = 16
_NBUF = 4
_NOBUF = 2
_RUNROLL = 4
_SUBS = ((0, 104), (104, 96))


def _rsqrt_scalar(x):
    i = lax.bitcast_convert_type(x, jnp.int32)
    y = lax.bitcast_convert_type(jnp.int32(0x5F3759DF) - (i >> 1),
                                 jnp.float32)
    for _ in range(3):
        y = y * (1.5 - 0.5 * x * y * y)
    return y


def _parity_permute(a):
    s = a.shape[:-1]
    return a.reshape(*s, 2, 16, 2).transpose(
        *range(len(s)), len(s), len(s) + 2, len(s) + 1).reshape(*s, 64)


def _make_sc_kernel(n_batch, seq_len, n_workers):
    bat_per_w = n_batch // n_workers
    rows_per_w = bat_per_w * seq_len
    n_chunks = bat_per_w
    assert n_chunks % _NBUF == 0
    nq = _D // _LANES
    mesh = plsc.VectorSubcoreMesh(core_axis_name="c", subcore_axis_name="s")

    @functools.partial(
        pl.kernel,
        out_type=jax.ShapeDtypeStruct((n_batch, seq_len, 2 * _D),
                                      jnp.float32),
        mesh=mesh,
        compiler_params=pltpu.CompilerParams(
            needs_layout_passes=False, use_tc_tiling_on_sc=False),
        scratch_types=[
            pltpu.VMEM((rows_per_w,), jnp.int32),
            pltpu.VMEM((_NBUF, seq_len, _D), jnp.bfloat16),
            pltpu.VMEM((_NOBUF, seq_len, _D), jnp.float32),
            pltpu.VMEM((seq_len, _D), jnp.float32),
            pltpu.VMEM((_D,), jnp.float32),
            pltpu.VMEM((_D,), jnp.float32),
        ]
        + [pltpu.SemaphoreType.DMA] * (_NBUF + _NOBUF),
    )
    def sc_kernel(ids_hbm, table_hbm, pe_hbm, w_hbm, b_hbm, out_hbm,
                  idx_v, rows_v, out_v, pe_v, w_v, b_v, *sems):
        gsem = sems[:_NBUF]
        osem = sems[_NBUF:]
        wid = lax.axis_index("s") * 2 + lax.axis_index("c")
        row_base = wid * rows_per_w
        bat_base = wid * bat_per_w

        pltpu.sync_copy(ids_hbm.at[pl.ds(row_base, rows_per_w)], idx_v)
        pltpu.sync_copy(pe_hbm, pe_v)
        pltpu.sync_copy(w_hbm, w_v)
        pltpu.sync_copy(b_hbm, b_v)

        wq = [w_v[pl.ds(q * _LANES, _LANES)] for q in range(nq)]
        bq = [b_v[pl.ds(q * _LANES, _LANES)] for q in range(nq)]
        ev2 = lax.iota(jnp.int32, _LANES) * 2
        colq = [ev2 + (q // 2) * 32 + (q % 2) for q in range(nq)]

        def fire_gather(chunk, buf):
            for off, cnt in _SUBS:
                pltpu.async_copy(
                    table_hbm.at[idx_v.at[pl.ds(chunk * seq_len + off, cnt)]],
                    rows_v.at[buf].at[pl.ds(off, cnt)],
                    gsem[buf],
                )

        def wait_gather(buf):
            pltpu.make_async_copy(
                table_hbm.at[pl.ds(0, seq_len)], rows_v.at[buf], gsem[buf]
            ).wait()

        def fire_out(chunk, slot):
            pltpu.async_copy(
                out_v.at[slot],
                out_hbm.at[bat_base + chunk].at[:, pl.ds(0, _D)],
                osem[slot])

        def wait_out(slot):
            pltpu.make_async_copy(
                out_v.at[slot], out_hbm.at[0].at[:, pl.ds(0, _D)],
                osem[slot]).wait()

        def compute(buf, slot):
            rv = rows_v.at[buf]
            ov = out_v.at[slot]

            @plsc.parallel_loop(0, seq_len, 1, unroll=_RUNROLL)
            def _row(r):
                x = []
                for g in range(2):
                    xb = rv[r, pl.ds(32 * g, 32)]
                    ev, od = plsc.unpack(xb, format=plsc.PackFormat.INTERLEAVED)
                    x.append(ev.astype(jnp.float32))
                    x.append(od.astype(jnp.float32))
                x = [x[q] + pe_v[r, pl.ds(q * _LANES, _LANES)]
                     for q in range(nq)]
                t = (x[0] + x[1]) + (x[2] + x[3])
                u = (x[0] * x[0] + x[1] * x[1]) + (
                    x[2] * x[2] + x[3] * x[3])
                mean = jnp.sum(t) * (1.0 / _D)
                var = jnp.sum(u) * (1.0 / _D) - mean * mean
                rstd = _rsqrt_scalar(var + 1e-5)
                rsplat = jnp.full((_LANES,), 0, jnp.int32) + r
                for q in range(nq):
                    y = (x[q] - mean) * rstd * wq[q] + bq[q]
                    plsc.store_scatter(ov, [rsplat, colq[q]], y)

        for c in range(_NBUF - 1):
            fire_gather(c, c)

        def outer(c4, _):
            for i in range(_NBUF):
                chunk = c4 * _NBUF + i
                slot = i % _NOBUF
                wait_gather(i)
                if i < 2:

                    @pl.when(c4 > 0)
                    def _():
                        wait_out(slot)
                else:
                    wait_out(slot)
                compute(i, slot)
                fire_out(chunk, slot)
                nb = (i + _NBUF - 1) % _NBUF
                if i == 0:
                    fire_gather(chunk + _NBUF - 1, nb)
                else:

                    @pl.when(c4 < n_chunks // _NBUF - 1)
                    def _():
                        fire_gather(chunk + _NBUF - 1, nb)

            return 0

        lax.fori_loop(0, n_chunks // _NBUF, outer, 0)
        for s_ in range(_NOBUF):
            wait_out(s_)

    return sc_kernel


@jax.jit
def kernel(input_ids, word_table, pe, ln_weight, ln_bias):
    b, s = input_ids.shape
    n_workers = 32
    assert b % (n_workers * _NBUF) == 0

    ids_flat = input_ids.reshape(b * s).astype(jnp.int32) * 2
    tab = jnp.pad(word_table.astype(jnp.bfloat16), ((0, 0), (0, _D)))
    tab = tab.reshape(2 * word_table.shape[0], _D)
    pe_p = _parity_permute(pe[:s].astype(jnp.float32))
    w_p = _parity_permute(ln_weight.astype(jnp.float32))
    b_p = _parity_permute(ln_bias.astype(jnp.float32))

    sc = _make_sc_kernel(b, s, n_workers)
    out = sc(ids_flat, tab, pe_p, w_p, b_p)
    return out[:, :, :_D]


# speedup vs baseline: 1.8703x; 1.8703x over previous
import functools
import math

import jax
import jax.numpy as jnp
from jax import lax
from jax.experimental import pallas as pl
from jax.experimental.pallas import tpu as pltpu
from jax.experimental.pallas import tpu_sc as plsc

_D = 64
_LANES = 16
_NBUF = 4
_RUNROLL = 4
_SUBS = ((0, 104), (104, 96))


def _rsqrt_scalar(x):
    i = lax.bitcast_convert_type(x, jnp.int32)
    y = lax.bitcast_convert_type(jnp.int32(0x5F3759DF) - (i >> 1),
                                 jnp.float32)
    for _ in range(3):
        y = y * (1.5 - 0.5 * x * y * y)
    return y


def _make_sc_kernel(n_batch, seq_len, n_workers):
    bat_per_w = n_batch // n_workers
    rows_per_w = bat_per_w * seq_len
    n_chunks = bat_per_w
    assert n_chunks % _NBUF == 0
    nq = _D // _LANES
    mesh = plsc.VectorSubcoreMesh(core_axis_name="c", subcore_axis_name="s")

    @functools.partial(
        pl.kernel,
        out_type=jax.ShapeDtypeStruct((n_batch, seq_len, 2 * _D),
                                      jnp.float32),
        mesh=mesh,
        compiler_params=pltpu.CompilerParams(
            needs_layout_passes=False, use_tc_tiling_on_sc=False),
        scratch_types=[
            pltpu.VMEM((rows_per_w,), jnp.int32),
            pltpu.VMEM((_NBUF, seq_len, _D), jnp.float32),
            pltpu.VMEM((seq_len, _D), jnp.float32),
            pltpu.VMEM((_D,), jnp.float32),
            pltpu.VMEM((_D,), jnp.float32),
        ]
        + [pltpu.SemaphoreType.DMA] * (2 * _NBUF),
    )
    def sc_kernel(ids_hbm, table_hbm, pe_hbm, w_hbm, b_hbm, out_hbm,
                  idx_v, rows_v, pe_v, w_v, b_v, *sems):
        gsem = sems[:_NBUF]
        osem = sems[_NBUF:]
        wid = lax.axis_index("s") * 2 + lax.axis_index("c")
        row_base = wid * rows_per_w
        bat_base = wid * bat_per_w

        pltpu.sync_copy(ids_hbm.at[pl.ds(row_base, rows_per_w)], idx_v)
        pltpu.sync_copy(pe_hbm, pe_v)
        pltpu.sync_copy(w_hbm, w_v)
        pltpu.sync_copy(b_hbm, b_v)

        wq = [w_v[pl.ds(q * _LANES, _LANES)] for q in range(nq)]
        bq = [b_v[pl.ds(q * _LANES, _LANES)] for q in range(nq)]

        def fire_gather(chunk, buf):
            for off, cnt in _SUBS:
                pltpu.async_copy(
                    table_hbm.at[idx_v.at[pl.ds(chunk * seq_len + off, cnt)]],
                    rows_v.at[buf].at[pl.ds(off, cnt)],
                    gsem[buf],
                )

        def wait_gather(buf):
            pltpu.make_async_copy(
                table_hbm.at[pl.ds(0, seq_len)], rows_v.at[buf], gsem[buf]
            ).wait()

        def fire_out(chunk, buf):
            pltpu.async_copy(
                rows_v.at[buf],
                out_hbm.at[bat_base + chunk].at[:, pl.ds(0, _D)],
                osem[buf])

        def wait_out(buf):
            pltpu.make_async_copy(
                rows_v.at[buf], out_hbm.at[0].at[:, pl.ds(0, _D)],
                osem[buf]).wait()

        def compute(chunk, buf):
            rv = rows_v.at[buf]

            @plsc.parallel_loop(0, seq_len, 1, unroll=_RUNROLL)
            def _row(r):
                x = [
                    rv[r, pl.ds(q * _LANES, _LANES)]
                    + pe_v[r, pl.ds(q * _LANES, _LANES)]
                    for q in range(nq)
                ]
                t = (x[0] + x[1]) + (x[2] + x[3])
                u = (x[0] * x[0] + x[1] * x[1]) + (
                    x[2] * x[2] + x[3] * x[3])
                mean = jnp.sum(t) * (1.0 / _D)
                var = jnp.sum(u) * (1.0 / _D) - mean * mean
                rstd = _rsqrt_scalar(var + 1e-5)
                for q in range(nq):
                    rv[r, pl.ds(q * _LANES, _LANES)] = (
                        (x[q] - mean) * rstd * wq[q] + bq[q])

        for c in range(_NBUF - 1):
            fire_gather(c, c)

        def outer(c4, _):
            for i in range(_NBUF):
                chunk = c4 * _NBUF + i
                wait_gather(i)
                compute(chunk, i)
                fire_out(chunk, i)
                nb = (i + _NBUF - 1) % _NBUF

                def prefetch():
                    wait_out(nb)
                    fire_gather(chunk + _NBUF - 1, nb)

                if i == 0:

                    @pl.when(c4 == 0)
                    def _():
                        fire_gather(_NBUF - 1, nb)

                    @pl.when(c4 > 0)
                    def _():
                        prefetch()
                else:

                    @pl.when(c4 < n_chunks // _NBUF - 1)
                    def _():
                        prefetch()

            return 0

        lax.fori_loop(0, n_chunks // _NBUF, outer, 0)
        for b in range(_NBUF):
            wait_out(b)

    return sc_kernel


@jax.jit
def kernel(input_ids, word_table, pe, ln_weight, ln_bias):
    b, s = input_ids.shape
    n_workers = 32
    assert b % (n_workers * _NBUF) == 0

    ids_flat = input_ids.reshape(b * s).astype(jnp.int32) * 2
    tab = jnp.pad(word_table.astype(jnp.float32), ((0, 0), (0, _D)))
    tab = tab.reshape(2 * word_table.shape[0], _D)
    pe_s = pe[:s].astype(jnp.float32)

    sc = _make_sc_kernel(b, s, n_workers)
    out = sc(ids_flat, tab, pe_s,
             ln_weight.astype(jnp.float32), ln_bias.astype(jnp.float32))
    return out[:, :, :_D]
